# Initial kernel scaffold; baseline (speedup 1.0000x reference)
import jax
import jax.numpy as jnp
from jax.experimental import pallas as pl


def kernel(node_type, edge_index, edge_attr, bond_dist, bond_vec, emb_table, Wd1, bd1, Wd2, bd2, Wd3, bd3, W_emb2, b_emb2, W_emb3, b_emb3, Wt0, Wt1, Wt2, Ws0, bs0, Ws1, bs1, ln_g, ln_b):
    raise NotImplementedError("write your pallas kernel here")


# speedup vs baseline: 35.1101x; 35.1101x over previous
import functools

import jax
import jax.numpy as jnp
from jax import lax
from jax.experimental import pallas as pl
from jax.experimental.pallas import tpu as pltpu
from jax.experimental.pallas import tpu_sc as plsc

N_NODES = 10000
N_EDGES = 160000
UNITS = 32
CUTOFF = 5.0

NC = 2
NS = 16
CH = 80
EPW = N_EDGES // NS
NPT = N_NODES // NS
NZB = 25
BE = 2000
BN = 1000


def _edge_kernel(ea_ref, bd_ref, bv_ref, wd1_ref, bd1_ref, wd2_ref, bd2_ref,
                 wd3_ref, bd3_ref, we3_ref, be3_ref,
                 wh_ref, basis_ref, ef_ref):
    ea = ea_ref[...]
    r = bd_ref[...]
    c = jnp.where(r <= CUTOFF, 0.5 * (jnp.cos(jnp.pi * r / CUTOFF) + 1.0), 0.0)

    def mm(x, w):
        return lax.dot_general(x, w, (((1,), (1,)), ((), ())),
                               preferred_element_type=jnp.float32,
                               precision=lax.Precision.HIGHEST)

    w1 = (mm(ea, wd1_ref[...]) + bd1_ref[...]) * c
    w2 = (mm(ea, wd2_ref[...]) + bd2_ref[...]) * c
    w3 = (mm(ea, wd3_ref[...]) + bd3_ref[...]) * c
    wh_ref[0] = jnp.concatenate([w1[:, :16], w2[:, :16], w3[:, :16]], axis=1)
    wh_ref[1] = jnp.concatenate([w1[:, 16:], w2[:, 16:], w3[:, 16:]], axis=1)

    v = bv_ref[...]
    inv = 1.0 / jnp.sqrt(jnp.sum(v * v, axis=1, keepdims=True))
    ev = v * inv
    e0 = ev[:, 0:1]
    e1 = ev[:, 1:2]
    e2 = ev[:, 2:3]
    z = jnp.zeros_like(e0)
    basis_ref[...] = jnp.concatenate(
        [e0, e1, e2, e0 * e0, e0 * e1, e0 * e2, e1 * e1, e1 * e2, e2 * e2,
         z, z, z, z, z, z, z], axis=1)

    ef_ref[...] = mm(ea, we3_ref[...]) + be3_ref[...]


def _edge_precompute(edge_attr, bond_dist, bond_vec, Wd1, bd1, Wd2, bd2,
                     Wd3, bd3, W_emb3, b_emb3):
    grid = (N_EDGES // BE,)
    wspec = pl.BlockSpec((32, 32), lambda i: (0, 0))
    bspec = pl.BlockSpec((1, 32), lambda i: (0, 0))
    return pl.pallas_call(
        _edge_kernel,
        grid=grid,
        in_specs=[
            pl.BlockSpec((BE, 32), lambda i: (i, 0)),
            pl.BlockSpec((BE, 1), lambda i: (i, 0)),
            pl.BlockSpec((BE, 3), lambda i: (i, 0)),
            wspec, bspec, wspec, bspec, wspec, bspec, wspec, bspec,
        ],
        out_specs=[
            pl.BlockSpec((2, BE, 48), lambda i: (0, i, 0)),
            pl.BlockSpec((BE, 16), lambda i: (i, 0)),
            pl.BlockSpec((BE, 32), lambda i: (i, 0)),
        ],
        out_shape=[
            jax.ShapeDtypeStruct((2, N_EDGES, 48), jnp.float32),
            jax.ShapeDtypeStruct((N_EDGES, 16), jnp.float32),
            jax.ShapeDtypeStruct((N_EDGES, 32), jnp.float32),
        ],
    )(edge_attr, bond_dist[:, None], bond_vec, Wd1, bd1[None, :], Wd2,
      bd2[None, :], Wd3, bd3[None, :], W_emb3, b_emb3[None, :])


def _node_kernel(nt_ref, emb_ref, wa_ref, wb_ref, b2_ref, u_ref, v_ref):
    nt = nt_ref[...]
    iota = lax.broadcasted_iota(jnp.int32, (BN, 128), 1)
    oh = (nt == iota).astype(jnp.float32)

    def mm_t(x, w):
        return lax.dot_general(x, w, (((1,), (1,)), ((), ())),
                               preferred_element_type=jnp.float32,
                               precision=lax.Precision.HIGHEST)

    def mm(x, w):
        return lax.dot_general(x, w, (((1,), (0,)), ((), ())),
                               preferred_element_type=jnp.float32,
                               precision=lax.Precision.HIGHEST)

    ma = mm_t(emb_ref[...], wa_ref[...])
    mb = mm_t(emb_ref[...], wb_ref[...])
    u = mm(oh, ma)
    v = mm(oh, mb) + b2_ref[...]
    u_ref[0] = u[:, :16]
    u_ref[1] = u[:, 16:]
    v_ref[0] = v[:, :16]
    v_ref[1] = v[:, 16:]


def _node_precompute(node_type, emb_table, W_emb2, b_emb2):
    emb_pad = jnp.zeros((128, 32), jnp.float32).at[:emb_table.shape[0]].set(
        emb_table)
    wa = W_emb2[:, :UNITS]
    wb = W_emb2[:, UNITS:]
    grid = (N_NODES // BN,)
    full = lambda s: pl.BlockSpec(s, lambda i: tuple(0 for _ in s))
    return pl.pallas_call(
        _node_kernel,
        grid=grid,
        in_specs=[
            pl.BlockSpec((BN, 1), lambda i: (i, 0)),
            full((128, 32)), full((32, 32)), full((32, 32)), full((1, 32)),
        ],
        out_specs=[
            pl.BlockSpec((2, BN, 16), lambda i: (0, i, 0)),
            pl.BlockSpec((2, BN, 16), lambda i: (0, i, 0)),
        ],
        out_shape=[
            jax.ShapeDtypeStruct((2, N_NODES, 16), jnp.float32),
            jax.ShapeDtypeStruct((2, N_NODES, 16), jnp.float32),
        ],
    )(node_type[:, None], emb_pad, wa, wb, b_emb2[None, :])


def _sc_scatter(edge_index, Wh, basis, U2, V2):
    mesh = plsc.VectorSubcoreMesh(core_axis_name="c", subcore_axis_name="s")

    @functools.partial(
        pl.kernel,
        out_type=jax.ShapeDtypeStruct((NC, N_NODES, 160), jnp.float32),
        mesh=mesh,
        compiler_params=pltpu.CompilerParams(use_tc_tiling_on_sc=False),
        scratch_types=[
            pltpu.VMEM_SHARED((N_NODES, 160), jnp.float32),
            pltpu.VMEM((CH,), jnp.int32),
            pltpu.VMEM((CH,), jnp.int32),
            pltpu.VMEM((CH, 48), jnp.float32),
            pltpu.VMEM((CH, 16), jnp.float32),
            pltpu.VMEM((CH, 16), jnp.float32),
            pltpu.VMEM((CH, 16), jnp.float32),
            pltpu.VMEM((CH, 160), jnp.float32),
            pltpu.VMEM((NZB, 160), jnp.float32),
            pltpu.SemaphoreType.DMA,
        ],
    )
    def sc_fn(ei, wh, bas, u2, v2, out, acc, src_i, dst_i, wc, bc, ur, vr,
              pay, zb, sem):
        c = lax.axis_index("c")
        s = lax.axis_index("s")
        base_n = s * NPT

        def zrow(i, _):
            for k in range(10):
                zb[i, pl.ds(16 * k, 16)] = jnp.zeros((16,), jnp.float32)
            return 0
        lax.fori_loop(0, NZB, zrow, 0)

        def zcopy(j, _):
            pltpu.sync_copy(zb, acc.at[pl.ds(base_n + NZB * j, NZB)])
            return 0
        lax.fori_loop(0, NPT // NZB, zcopy, 0)
        plsc.subcore_barrier()

        def chunk_body(i, _):
            e0 = s * EPW + i * CH
            pltpu.sync_copy(ei.at[0, pl.ds(e0, CH)], src_i)
            pltpu.sync_copy(ei.at[1, pl.ds(e0, CH)], dst_i)
            cp_u = pltpu.async_copy(u2.at[c].at[src_i], ur, sem)
            cp_v = pltpu.async_copy(v2.at[c].at[dst_i], vr, sem)
            pltpu.sync_copy(wh.at[c, pl.ds(e0, CH)], wc)
            pltpu.sync_copy(bas.at[pl.ds(e0, CH)], bc)
            cp_u.wait()
            cp_v.wait()

            def edge_body(e, _):
                zij = ur[e, :] + vr[e, :]
                g1 = zij * wc[e, pl.ds(0, 16)]
                g2 = zij * wc[e, pl.ds(16, 16)]
                g3 = zij * wc[e, pl.ds(32, 16)]
                bb = bc[e, :]
                pay[e, pl.ds(0, 16)] = g1
                for d in range(3):
                    pay[e, pl.ds(16 + 16 * d, 16)] = g2 * bb[d]
                for k in range(6):
                    pay[e, pl.ds(64 + 16 * k, 16)] = g3 * bb[3 + k]
                return 0
            lax.fori_loop(0, CH, edge_body, 0)

            pltpu.sync_copy(pay, acc.at[dst_i], add=True)
            return 0
        lax.fori_loop(0, EPW // CH, chunk_body, 0)
        plsc.subcore_barrier()

        def drain(j, _):
            r0 = base_n + NZB * j
            pltpu.sync_copy(acc.at[pl.ds(r0, NZB)], zb)
            pltpu.sync_copy(zb, out.at[c, pl.ds(r0, NZB)])
            return 0
        lax.fori_loop(0, NPT // NZB, drain, 0)

    return sc_fn(edge_index, Wh, basis, U2, V2)


def _finish_kernel(g0_ref, g1h_ref, lng_ref, lnb_ref, ws0_ref, bs0_ref,
                   ws1_ref, bs1_ref, wt0_ref, wt1_ref, wt2_ref, *o_refs):
    gh0 = g0_ref[...]
    gh1 = g1h_ref[...]

    def comp(k):
        return jnp.concatenate(
            [gh0[:, 16 * k:16 * k + 16], gh1[:, 16 * k:16 * k + 16]], axis=1)

    G1 = comp(0)
    G2 = [comp(1), comp(2), comp(3)]
    P = [comp(4 + i) for i in range(6)]

    trP = P[0] + P[3] + P[5]
    Pn2 = (P[0] * P[0] + P[3] * P[3] + P[5] * P[5]
           + 2.0 * (P[1] * P[1] + P[2] * P[2] + P[4] * P[4]))
    nrm = (3.0 * G1 * G1 + 2.0 * (G2[0] * G2[0] + G2[1] * G2[1]
                                  + G2[2] * G2[2]) + Pn2 - trP * trP / 3.0)
    mu = jnp.mean(nrm, axis=1, keepdims=True)
    var = jnp.mean((nrm - mu) ** 2, axis=1, keepdims=True)
    nrm = (nrm - mu) / jnp.sqrt(var + 1e-5) * lng_ref[...] + lnb_ref[...]

    def mm_t(x, w):
        return lax.dot_general(x, w, (((1,), (1,)), ((), ())),
                               preferred_element_type=jnp.float32,
                               precision=lax.Precision.HIGHEST)

    h = mm_t(nrm, ws0_ref[...]) + bs0_ref[...]
    h = h * jax.nn.sigmoid(h)
    h = mm_t(h, ws1_ref[...]) + bs1_ref[...]
    h = h * jax.nn.sigmoid(h)
    n0 = h[:, 0:32]
    n1 = h[:, 32:64]
    n2 = h[:, 64:96]

    A0 = mm_t(G1, wt0_ref[...])
    w0 = mm_t(G2[0], wt1_ref[...])
    w1 = mm_t(G2[1], wt1_ref[...])
    w2 = mm_t(G2[2], wt1_ref[...])
    Pp = [mm_t(P[k], wt2_ref[...]) for k in range(6)]
    t3 = (Pp[0] + Pp[3] + Pp[5]) / 3.0

    diag = n0 * A0
    o_refs[0][...] = diag + n2 * (Pp[0] - t3)
    o_refs[1][...] = n2 * Pp[1] - n1 * w2
    o_refs[2][...] = n2 * Pp[2] + n1 * w1
    o_refs[3][...] = n2 * Pp[1] + n1 * w2
    o_refs[4][...] = diag + n2 * (Pp[3] - t3)
    o_refs[5][...] = n2 * Pp[4] - n1 * w0
    o_refs[6][...] = n2 * Pp[2] - n1 * w1
    o_refs[7][...] = n2 * Pp[4] + n1 * w0
    o_refs[8][...] = diag + n2 * (Pp[5] - t3)


def _node_finish(Gh, ln_g, ln_b, Ws0, bs0, Ws1, bs1, Wt0, Wt1, Wt2):
    perm = jnp.asarray([3 * c + k for k in range(3) for c in range(32)],
                       dtype=jnp.int32)
    ws1p = Ws1[perm, :]
    bs1p = bs1[perm]
    grid = (N_NODES // BN,)
    full = lambda s: pl.BlockSpec(s, lambda i: tuple(0 for _ in s))
    outs = pl.pallas_call(
        _finish_kernel,
        grid=grid,
        in_specs=[
            pl.BlockSpec((BN, 160), lambda i: (i, 0)),
            pl.BlockSpec((BN, 160), lambda i: (i, 0)),
            full((1, 32)), full((1, 32)), full((64, 32)), full((1, 64)),
            full((96, 64)), full((1, 96)), full((32, 32)), full((32, 32)),
            full((32, 32)),
        ],
        out_specs=[pl.BlockSpec((BN, 32), lambda i: (i, 0))] * 9,
        out_shape=[jax.ShapeDtypeStruct((N_NODES, 32), jnp.float32)] * 9,
    )(Gh[0], Gh[1], ln_g[None, :], ln_b[None, :], Ws0, bs0[None, :],
      ws1p, bs1p[None, :], Wt0, Wt1, Wt2)
    return jnp.stack(outs, axis=-1).reshape(N_NODES, UNITS, 3, 3)


def kernel(node_type, edge_index, edge_attr, bond_dist, bond_vec, emb_table,
           Wd1, bd1, Wd2, bd2, Wd3, bd3, W_emb2, b_emb2, W_emb3, b_emb3,
           Wt0, Wt1, Wt2, Ws0, bs0, Ws1, bs1, ln_g, ln_b):
    Wh, basis, edge_feat = _edge_precompute(
        edge_attr, bond_dist, bond_vec, Wd1, bd1, Wd2, bd2, Wd3, bd3,
        W_emb3, b_emb3)
    U2, V2 = _node_precompute(node_type, emb_table, W_emb2, b_emb2)
    Gh = _sc_scatter(edge_index, Wh, basis, U2, V2)
    X = _node_finish(Gh, ln_g, ln_b, Ws0, bs0, Ws1, bs1, Wt0, Wt1, Wt2)
    return X, edge_feat


# speedup vs baseline: 38.9514x; 1.1094x over previous
import functools

import jax
import jax.numpy as jnp
from jax import lax
from jax.experimental import pallas as pl
from jax.experimental.pallas import tpu as pltpu
from jax.experimental.pallas import tpu_sc as plsc

N_NODES = 10000
N_EDGES = 160000
UNITS = 32
CUTOFF = 5.0

NC = 2
NS = 16
CH = 80
EPW = N_EDGES // NS
NPT = N_NODES // NS
NZB = 25
BE = 2000
BN = 1000


def _edge_kernel(ea_ref, bd_ref, bv_ref, wd1_ref, bd1_ref, wd2_ref, bd2_ref,
                 wd3_ref, bd3_ref, we3_ref, be3_ref,
                 wh_ref, basis_ref, ef_ref):
    ea = ea_ref[...]
    r = bd_ref[...]
    c = jnp.where(r <= CUTOFF, 0.5 * (jnp.cos(jnp.pi * r / CUTOFF) + 1.0), 0.0)

    def mm(x, w):
        return lax.dot_general(x, w, (((1,), (1,)), ((), ())),
                               preferred_element_type=jnp.float32,
                               precision=lax.Precision.HIGHEST)

    w1 = (mm(ea, wd1_ref[...]) + bd1_ref[...]) * c
    w2 = (mm(ea, wd2_ref[...]) + bd2_ref[...]) * c
    w3 = (mm(ea, wd3_ref[...]) + bd3_ref[...]) * c
    wh_ref[0] = jnp.concatenate([w1[:, :16], w2[:, :16], w3[:, :16]], axis=1)
    wh_ref[1] = jnp.concatenate([w1[:, 16:], w2[:, 16:], w3[:, 16:]], axis=1)

    v = bv_ref[...]
    inv = 1.0 / jnp.sqrt(jnp.sum(v * v, axis=1, keepdims=True))
    ev = v * inv
    e0 = ev[:, 0:1]
    e1 = ev[:, 1:2]
    e2 = ev[:, 2:3]
    z = jnp.zeros_like(e0)
    basis_ref[...] = jnp.concatenate(
        [e0, e1, e2, e0 * e0, e0 * e1, e0 * e2, e1 * e1, e1 * e2, e2 * e2,
         z, z, z, z, z, z, z], axis=1)

    ef_ref[...] = mm(ea, we3_ref[...]) + be3_ref[...]


def _edge_precompute(edge_attr, bond_dist, bond_vec, Wd1, bd1, Wd2, bd2,
                     Wd3, bd3, W_emb3, b_emb3):
    grid = (N_EDGES // BE,)
    wspec = pl.BlockSpec((32, 32), lambda i: (0, 0))
    bspec = pl.BlockSpec((1, 32), lambda i: (0, 0))
    return pl.pallas_call(
        _edge_kernel,
        grid=grid,
        in_specs=[
            pl.BlockSpec((BE, 32), lambda i: (i, 0)),
            pl.BlockSpec((BE, 1), lambda i: (i, 0)),
            pl.BlockSpec((BE, 3), lambda i: (i, 0)),
            wspec, bspec, wspec, bspec, wspec, bspec, wspec, bspec,
        ],
        out_specs=[
            pl.BlockSpec((2, BE, 48), lambda i: (0, i, 0)),
            pl.BlockSpec((BE, 16), lambda i: (i, 0)),
            pl.BlockSpec((BE, 32), lambda i: (i, 0)),
        ],
        out_shape=[
            jax.ShapeDtypeStruct((2, N_EDGES, 48), jnp.float32),
            jax.ShapeDtypeStruct((N_EDGES, 16), jnp.float32),
            jax.ShapeDtypeStruct((N_EDGES, 32), jnp.float32),
        ],
    )(edge_attr, bond_dist[:, None], bond_vec, Wd1, bd1[None, :], Wd2,
      bd2[None, :], Wd3, bd3[None, :], W_emb3, b_emb3[None, :])


def _node_kernel(nt_ref, emb_ref, wa_ref, wb_ref, b2_ref, u_ref, v_ref):
    nt = nt_ref[...]
    iota = lax.broadcasted_iota(jnp.int32, (BN, 128), 1)
    oh = (nt == iota).astype(jnp.float32)

    def mm_t(x, w):
        return lax.dot_general(x, w, (((1,), (1,)), ((), ())),
                               preferred_element_type=jnp.float32,
                               precision=lax.Precision.HIGHEST)

    def mm(x, w):
        return lax.dot_general(x, w, (((1,), (0,)), ((), ())),
                               preferred_element_type=jnp.float32,
                               precision=lax.Precision.HIGHEST)

    ma = mm_t(emb_ref[...], wa_ref[...])
    mb = mm_t(emb_ref[...], wb_ref[...])
    u = mm(oh, ma)
    v = mm(oh, mb) + b2_ref[...]
    u_ref[0] = u[:, :16]
    u_ref[1] = u[:, 16:]
    v_ref[0] = v[:, :16]
    v_ref[1] = v[:, 16:]


def _node_precompute(node_type, emb_table, W_emb2, b_emb2):
    emb_pad = jnp.zeros((128, 32), jnp.float32).at[:emb_table.shape[0]].set(
        emb_table)
    wa = W_emb2[:, :UNITS]
    wb = W_emb2[:, UNITS:]
    grid = (N_NODES // BN,)
    full = lambda s: pl.BlockSpec(s, lambda i: tuple(0 for _ in s))
    return pl.pallas_call(
        _node_kernel,
        grid=grid,
        in_specs=[
            pl.BlockSpec((BN, 1), lambda i: (i, 0)),
            full((128, 32)), full((32, 32)), full((32, 32)), full((1, 32)),
        ],
        out_specs=[
            pl.BlockSpec((2, BN, 16), lambda i: (0, i, 0)),
            pl.BlockSpec((2, BN, 16), lambda i: (0, i, 0)),
        ],
        out_shape=[
            jax.ShapeDtypeStruct((2, N_NODES, 16), jnp.float32),
            jax.ShapeDtypeStruct((2, N_NODES, 16), jnp.float32),
        ],
    )(node_type[:, None], emb_pad, wa, wb, b_emb2[None, :])


def _sc_scatter(edge_index, Wh, basis, U2, V2):
    mesh = plsc.VectorSubcoreMesh(core_axis_name="c", subcore_axis_name="s")
    n_chunks = EPW // CH

    vset = lambda: [pltpu.VMEM((CH,), jnp.int32),
                    pltpu.VMEM((CH,), jnp.int32),
                    pltpu.VMEM((CH, 48), jnp.float32),
                    pltpu.VMEM((CH, 16), jnp.float32),
                    pltpu.VMEM((CH, 16), jnp.float32),
                    pltpu.VMEM((CH, 16), jnp.float32)]

    @functools.partial(
        pl.kernel,
        out_type=jax.ShapeDtypeStruct((NC, N_NODES, 160), jnp.float32),
        mesh=mesh,
        compiler_params=pltpu.CompilerParams(use_tc_tiling_on_sc=False),
        scratch_types=[
            pltpu.VMEM_SHARED((N_NODES, 160), jnp.float32),
            vset(), vset(),
            pltpu.VMEM((CH, 160), jnp.float32),
            pltpu.SemaphoreType.DMA, pltpu.SemaphoreType.DMA,
            pltpu.SemaphoreType.DMA, pltpu.SemaphoreType.DMA,
        ],
    )
    def sc_fn(ei, wh, bas, u2, v2, out, acc, set0, set1, pay,
              sl0, sl1, sg0, sg1):
        c = lax.axis_index("c")
        s = lax.axis_index("s")
        base_n = s * NPT
        sets = (set0, set1)
        sem_l = (sl0, sl1)
        sem_g = (sg0, sg1)

        def zrow(i, _):
            for k in range(10):
                pay[i, pl.ds(16 * k, 16)] = jnp.zeros((16,), jnp.float32)
            return 0
        lax.fori_loop(0, CH, zrow, 0)

        def zcopy(j, _):
            pltpu.sync_copy(pay, acc.at[pl.ds(base_n + CH * j, CH)])
            return 0
        lax.fori_loop(0, NPT // CH, zcopy, 0)
        rem = NPT - (NPT // CH) * CH
        if rem:
            pltpu.sync_copy(pay.at[pl.ds(0, rem)],
                            acc.at[pl.ds(base_n + (NPT // CH) * CH, rem)])
        plsc.subcore_barrier()

        def e_of(i):
            return s * EPW + jnp.minimum(i, n_chunks - 1) * CH

        def fire_linear(i, p):
            src_i, dst_i, wc, bc, _, _ = sets[p]
            e0 = e_of(i)
            pltpu.async_copy(ei.at[0, pl.ds(e0, CH)], src_i, sem_l[p])
            pltpu.async_copy(ei.at[1, pl.ds(e0, CH)], dst_i, sem_l[p])
            pltpu.async_copy(wh.at[c, pl.ds(e0, CH)], wc, sem_l[p])
            pltpu.async_copy(bas.at[pl.ds(e0, CH)], bc, sem_l[p])

        def wait_linear(p):
            src_i, dst_i, wc, bc, _, _ = sets[p]
            e0 = s * EPW
            pltpu.make_async_copy(ei.at[0, pl.ds(e0, CH)], src_i, sem_l[p]).wait()
            pltpu.make_async_copy(ei.at[1, pl.ds(e0, CH)], dst_i, sem_l[p]).wait()
            pltpu.make_async_copy(wh.at[c, pl.ds(e0, CH)], wc, sem_l[p]).wait()
            pltpu.make_async_copy(bas.at[pl.ds(e0, CH)], bc, sem_l[p]).wait()

        def fire_gathers(p):
            src_i, dst_i, _, _, ur, vr = sets[p]
            pltpu.async_copy(u2.at[c].at[src_i], ur, sem_g[p])
            pltpu.async_copy(v2.at[c].at[dst_i], vr, sem_g[p])

        def wait_gathers(p):
            src_i, dst_i, _, _, ur, vr = sets[p]
            pltpu.make_async_copy(u2.at[c].at[src_i], ur, sem_g[p]).wait()
            pltpu.make_async_copy(v2.at[c].at[dst_i], vr, sem_g[p]).wait()

        def run_chunk(i, p):
            q = 1 - p
            src_i, dst_i, wc, bc, ur, vr = sets[p]
            wait_linear(q)
            fire_gathers(q)
            wait_gathers(p)

            def edge_body(e, _):
                zij = ur[e, :] + vr[e, :]
                g1 = zij * wc[e, pl.ds(0, 16)]
                g2 = zij * wc[e, pl.ds(16, 16)]
                g3 = zij * wc[e, pl.ds(32, 16)]
                bb = bc[e, :]
                pay[e, pl.ds(0, 16)] = g1
                for d in range(3):
                    pay[e, pl.ds(16 + 16 * d, 16)] = g2 * bb[d]
                for k in range(6):
                    pay[e, pl.ds(64 + 16 * k, 16)] = g3 * bb[3 + k]
                return 0
            lax.fori_loop(0, CH, edge_body, 0)

            pltpu.sync_copy(pay, acc.at[dst_i], add=True)
            fire_linear(i + 2, p)

        fire_linear(0, 0)
        fire_linear(1, 1)
        wait_linear(0)
        fire_gathers(0)

        def body2(k, _):
            run_chunk(2 * k, 0)
            run_chunk(2 * k + 1, 1)
            return 0
        lax.fori_loop(0, n_chunks // 2, body2, 0)
        run_chunk(n_chunks - 1, (n_chunks - 1) % 2)

        wait_linear(0)
        wait_gathers(1)
        plsc.subcore_barrier()

        def drain(j, _):
            r0 = base_n + CH * j
            pltpu.sync_copy(acc.at[pl.ds(r0, CH)], pay)
            pltpu.sync_copy(pay, out.at[c, pl.ds(r0, CH)])
            return 0
        lax.fori_loop(0, NPT // CH, drain, 0)
        if rem:
            r0 = base_n + (NPT // CH) * CH
            pltpu.sync_copy(acc.at[pl.ds(r0, rem)], pay.at[pl.ds(0, rem)])
            pltpu.sync_copy(pay.at[pl.ds(0, rem)], out.at[c, pl.ds(r0, rem)])

    return sc_fn(edge_index, Wh, basis, U2, V2)


def _finish_kernel(g0_ref, g1h_ref, lng_ref, lnb_ref, ws0_ref, bs0_ref,
                   ws1_ref, bs1_ref, wt0_ref, wt1_ref, wt2_ref, *o_refs):
    gh0 = g0_ref[...]
    gh1 = g1h_ref[...]

    def comp(k):
        return jnp.concatenate(
            [gh0[:, 16 * k:16 * k + 16], gh1[:, 16 * k:16 * k + 16]], axis=1)

    G1 = comp(0)
    G2 = [comp(1), comp(2), comp(3)]
    P = [comp(4 + i) for i in range(6)]

    trP = P[0] + P[3] + P[5]
    Pn2 = (P[0] * P[0] + P[3] * P[3] + P[5] * P[5]
           + 2.0 * (P[1] * P[1] + P[2] * P[2] + P[4] * P[4]))
    nrm = (3.0 * G1 * G1 + 2.0 * (G2[0] * G2[0] + G2[1] * G2[1]
                                  + G2[2] * G2[2]) + Pn2 - trP * trP / 3.0)
    mu = jnp.mean(nrm, axis=1, keepdims=True)
    var = jnp.mean((nrm - mu) ** 2, axis=1, keepdims=True)
    nrm = (nrm - mu) / jnp.sqrt(var + 1e-5) * lng_ref[...] + lnb_ref[...]

    def mm_t(x, w):
        return lax.dot_general(x, w, (((1,), (1,)), ((), ())),
                               preferred_element_type=jnp.float32,
                               precision=lax.Precision.HIGHEST)

    h = mm_t(nrm, ws0_ref[...]) + bs0_ref[...]
    h = h * jax.nn.sigmoid(h)
    h = mm_t(h, ws1_ref[...]) + bs1_ref[...]
    h = h * jax.nn.sigmoid(h)
    n0 = h[:, 0:32]
    n1 = h[:, 32:64]
    n2 = h[:, 64:96]

    A0 = mm_t(G1, wt0_ref[...])
    w0 = mm_t(G2[0], wt1_ref[...])
    w1 = mm_t(G2[1], wt1_ref[...])
    w2 = mm_t(G2[2], wt1_ref[...])
    Pp = [mm_t(P[k], wt2_ref[...]) for k in range(6)]
    t3 = (Pp[0] + Pp[3] + Pp[5]) / 3.0

    diag = n0 * A0
    o_refs[0][...] = diag + n2 * (Pp[0] - t3)
    o_refs[1][...] = n2 * Pp[1] - n1 * w2
    o_refs[2][...] = n2 * Pp[2] + n1 * w1
    o_refs[3][...] = n2 * Pp[1] + n1 * w2
    o_refs[4][...] = diag + n2 * (Pp[3] - t3)
    o_refs[5][...] = n2 * Pp[4] - n1 * w0
    o_refs[6][...] = n2 * Pp[2] - n1 * w1
    o_refs[7][...] = n2 * Pp[4] + n1 * w0
    o_refs[8][...] = diag + n2 * (Pp[5] - t3)


def _node_finish(Gh, ln_g, ln_b, Ws0, bs0, Ws1, bs1, Wt0, Wt1, Wt2):
    perm = jnp.asarray([3 * c + k for k in range(3) for c in range(32)],
                       dtype=jnp.int32)
    ws1p = Ws1[perm, :]
    bs1p = bs1[perm]
    grid = (N_NODES // BN,)
    full = lambda s: pl.BlockSpec(s, lambda i: tuple(0 for _ in s))
    outs = pl.pallas_call(
        _finish_kernel,
        grid=grid,
        in_specs=[
            pl.BlockSpec((BN, 160), lambda i: (i, 0)),
            pl.BlockSpec((BN, 160), lambda i: (i, 0)),
            full((1, 32)), full((1, 32)), full((64, 32)), full((1, 64)),
            full((96, 64)), full((1, 96)), full((32, 32)), full((32, 32)),
            full((32, 32)),
        ],
        out_specs=[pl.BlockSpec((BN, 32), lambda i: (i, 0))] * 9,
        out_shape=[jax.ShapeDtypeStruct((N_NODES, 32), jnp.float32)] * 9,
    )(Gh[0], Gh[1], ln_g[None, :], ln_b[None, :], Ws0, bs0[None, :],
      ws1p, bs1p[None, :], Wt0, Wt1, Wt2)
    return jnp.stack(outs, axis=-1).reshape(N_NODES, UNITS, 3, 3)


def kernel(node_type, edge_index, edge_attr, bond_dist, bond_vec, emb_table,
           Wd1, bd1, Wd2, bd2, Wd3, bd3, W_emb2, b_emb2, W_emb3, b_emb3,
           Wt0, Wt1, Wt2, Ws0, bs0, Ws1, bs1, ln_g, ln_b):
    Wh, basis, edge_feat = _edge_precompute(
        edge_attr, bond_dist, bond_vec, Wd1, bd1, Wd2, bd2, Wd3, bd3,
        W_emb3, b_emb3)
    U2, V2 = _node_precompute(node_type, emb_table, W_emb2, b_emb2)
    Gh = _sc_scatter(edge_index, Wh, basis, U2, V2)
    X = _node_finish(Gh, ln_g, ln_b, Ws0, bs0, Ws1, bs1, Wt0, Wt1, Wt2)
    return X, edge_feat


# speedup vs baseline: 63.3168x; 1.6255x over previous
import functools

import jax
import jax.numpy as jnp
from jax import lax
from jax.experimental import pallas as pl
from jax.experimental.pallas import tpu as pltpu
from jax.experimental.pallas import tpu_sc as plsc

N_NODES = 10000
N_EDGES = 160000
UNITS = 32
CUTOFF = 5.0

NC = 2
NS = 16
CH = 80
EPW = N_EDGES // NS
NPT = N_NODES // NS
BE = 3200
BN = 1000


def _edge_kernel(ea_ref, bd_ref, bv_ref, wd1_ref, bd1_ref, wd2_ref, bd2_ref,
                 wd3_ref, bd3_ref, we3_ref, be3_ref,
                 wb_ref, ef_ref):
    ea = ea_ref[...]

    def mm(x, w):
        return lax.dot_general(x, w, (((1,), (1,)), ((), ())),
                               preferred_element_type=jnp.float32,
                               precision=lax.Precision.HIGHEST)

    w1 = mm(ea, wd1_ref[...]) + bd1_ref[...]
    w2 = mm(ea, wd2_ref[...]) + bd2_ref[...]
    w3 = mm(ea, wd3_ref[...]) + bd3_ref[...]

    r = bd_ref[...]
    c = jnp.where(r <= CUTOFF, 0.5 * (jnp.cos(jnp.pi * r / CUTOFF) + 1.0), 0.0)

    v = bv_ref[...]
    inv = 1.0 / jnp.sqrt(jnp.sum(v * v, axis=0, keepdims=True))
    ev = v * inv
    sq = ev * ev
    evr = jnp.concatenate([ev[1:], ev[:1]], axis=0)
    cr = ev * evr
    comp = jnp.concatenate([ev, sq, cr, c, ev, ev], axis=0)
    basc = comp.T

    wb_ref[...] = jnp.concatenate(
        [w1[:, :16], w2[:, :16], w3[:, :16], basc,
         w1[:, 16:], w2[:, 16:], w3[:, 16:], basc], axis=1)

    ef_ref[...] = mm(ea, we3_ref[...]) + be3_ref[...]


def _edge_precompute(edge_attr, bond_dist, bond_vec, Wd1, bd1, Wd2, bd2,
                     Wd3, bd3, W_emb3, b_emb3):
    grid = (N_EDGES // BE,)
    wspec = pl.BlockSpec((32, 32), lambda i: (0, 0))
    bspec = pl.BlockSpec((1, 32), lambda i: (0, 0))
    return pl.pallas_call(
        _edge_kernel,
        grid=grid,
        in_specs=[
            pl.BlockSpec((BE, 32), lambda i: (i, 0)),
            pl.BlockSpec((1, BE), lambda i: (0, i)),
            pl.BlockSpec((3, BE), lambda i: (0, i)),
            wspec, bspec, wspec, bspec, wspec, bspec, wspec, bspec,
        ],
        out_specs=[
            pl.BlockSpec((BE, 128), lambda i: (i, 0)),
            pl.BlockSpec((BE, 32), lambda i: (i, 0)),
        ],
        out_shape=[
            jax.ShapeDtypeStruct((N_EDGES, 128), jnp.float32),
            jax.ShapeDtypeStruct((N_EDGES, 32), jnp.float32),
        ],
    )(edge_attr, bond_dist[None, :], bond_vec.T, Wd1, bd1[None, :], Wd2,
      bd2[None, :], Wd3, bd3[None, :], W_emb3, b_emb3[None, :])


def _node_kernel(nt_ref, emb_ref, wa_ref, wb_ref, b2_ref, u_ref, v_ref):
    nt = nt_ref[...]
    iota = lax.broadcasted_iota(jnp.int32, (BN, 128), 1)
    oh = (nt == iota).astype(jnp.float32)

    def mm_t(x, w):
        return lax.dot_general(x, w, (((1,), (1,)), ((), ())),
                               preferred_element_type=jnp.float32,
                               precision=lax.Precision.HIGHEST)

    def mm(x, w):
        return lax.dot_general(x, w, (((1,), (0,)), ((), ())),
                               preferred_element_type=jnp.float32,
                               precision=lax.Precision.HIGHEST)

    ma = mm_t(emb_ref[...], wa_ref[...])
    mb = mm_t(emb_ref[...], wb_ref[...])
    u = mm(oh, ma)
    v = mm(oh, mb) + b2_ref[...]
    u_ref[0] = u[:, :16]
    u_ref[1] = u[:, 16:]
    v_ref[0] = v[:, :16]
    v_ref[1] = v[:, 16:]


def _node_precompute(node_type, emb_table, W_emb2, b_emb2):
    emb_pad = jnp.zeros((128, 32), jnp.float32).at[:emb_table.shape[0]].set(
        emb_table)
    wa = W_emb2[:, :UNITS]
    wb = W_emb2[:, UNITS:]
    grid = (N_NODES // BN,)
    full = lambda s: pl.BlockSpec(s, lambda i: tuple(0 for _ in s))
    return pl.pallas_call(
        _node_kernel,
        grid=grid,
        in_specs=[
            pl.BlockSpec((BN, 1), lambda i: (i, 0)),
            full((128, 32)), full((32, 32)), full((32, 32)), full((1, 32)),
        ],
        out_specs=[
            pl.BlockSpec((2, BN, 16), lambda i: (0, i, 0)),
            pl.BlockSpec((2, BN, 16), lambda i: (0, i, 0)),
        ],
        out_shape=[
            jax.ShapeDtypeStruct((2, N_NODES, 16), jnp.float32),
            jax.ShapeDtypeStruct((2, N_NODES, 16), jnp.float32),
        ],
    )(node_type[:, None], emb_pad, wa, wb, b_emb2[None, :])


def _sc_scatter(edge_index, WB, U2, V2):
    mesh = plsc.VectorSubcoreMesh(core_axis_name="c", subcore_axis_name="s")
    n_chunks = EPW // CH

    vset = lambda: [pltpu.VMEM((CH,), jnp.int32),
                    pltpu.VMEM((CH,), jnp.int32),
                    pltpu.VMEM((CH, 64), jnp.float32),
                    pltpu.VMEM((CH, 16), jnp.float32),
                    pltpu.VMEM((CH, 16), jnp.float32)]

    @functools.partial(
        pl.kernel,
        out_type=jax.ShapeDtypeStruct((NC, N_NODES, 160), jnp.float32),
        mesh=mesh,
        compiler_params=pltpu.CompilerParams(use_tc_tiling_on_sc=False),
        scratch_types=[
            pltpu.VMEM_SHARED((N_NODES, 160), jnp.float32),
            vset(), vset(),
            pltpu.VMEM((CH, 160), jnp.float32),
            pltpu.SemaphoreType.DMA, pltpu.SemaphoreType.DMA,
            pltpu.SemaphoreType.DMA, pltpu.SemaphoreType.DMA,
        ],
    )
    def sc_fn(ei, wb, u2, v2, out, acc, set0, set1, pay,
              sl0, sl1, sg0, sg1):
        c = lax.axis_index("c")
        s = lax.axis_index("s")
        base_n = s * NPT
        sets = (set0, set1)
        sem_l = (sl0, sl1)
        sem_g = (sg0, sg1)

        def zrow(i, _):
            for k in range(10):
                pay[i, pl.ds(16 * k, 16)] = jnp.zeros((16,), jnp.float32)
            return 0
        lax.fori_loop(0, CH, zrow, 0)

        def zcopy(j, _):
            pltpu.sync_copy(pay, acc.at[pl.ds(base_n + CH * j, CH)])
            return 0
        lax.fori_loop(0, NPT // CH, zcopy, 0)
        rem = NPT - (NPT // CH) * CH
        if rem:
            pltpu.sync_copy(pay.at[pl.ds(0, rem)],
                            acc.at[pl.ds(base_n + (NPT // CH) * CH, rem)])
        plsc.subcore_barrier()

        def e_of(i):
            return s * EPW + jnp.minimum(i, n_chunks - 1) * CH

        def fire_linear(i, p):
            src_i, dst_i, wc, _, _ = sets[p]
            e0 = e_of(i)
            pltpu.async_copy(ei.at[0, pl.ds(e0, CH)], src_i, sem_l[p])
            pltpu.async_copy(ei.at[1, pl.ds(e0, CH)], dst_i, sem_l[p])
            pltpu.async_copy(wb.at[pl.ds(e0, CH), pl.ds(64 * c, 64)],
                             wc, sem_l[p])

        def wait_linear(p):
            src_i, dst_i, wc, _, _ = sets[p]
            e0 = s * EPW
            pltpu.make_async_copy(ei.at[0, pl.ds(e0, CH)], src_i, sem_l[p]).wait()
            pltpu.make_async_copy(ei.at[1, pl.ds(e0, CH)], dst_i, sem_l[p]).wait()
            pltpu.make_async_copy(wb.at[pl.ds(e0, CH), pl.ds(0, 64)],
                                  wc, sem_l[p]).wait()

        def fire_gathers(p):
            src_i, dst_i, _, ur, vr = sets[p]
            pltpu.async_copy(u2.at[c].at[src_i], ur, sem_g[p])
            pltpu.async_copy(v2.at[c].at[dst_i], vr, sem_g[p])

        def wait_gathers(p):
            src_i, dst_i, _, ur, vr = sets[p]
            pltpu.make_async_copy(u2.at[c].at[src_i], ur, sem_g[p]).wait()
            pltpu.make_async_copy(v2.at[c].at[dst_i], vr, sem_g[p]).wait()

        def run_chunk(i, p):
            q = 1 - p
            src_i, dst_i, wc, ur, vr = sets[p]
            wait_linear(q)
            fire_gathers(q)
            wait_gathers(p)

            def edge_body(e, _):
                bb = wc[e, pl.ds(48, 16)]
                zc = (ur[e, :] + vr[e, :]) * bb[9]
                g1 = zc * wc[e, pl.ds(0, 16)]
                g2 = zc * wc[e, pl.ds(16, 16)]
                g3 = zc * wc[e, pl.ds(32, 16)]
                pay[e, pl.ds(0, 16)] = g1
                for d in range(3):
                    pay[e, pl.ds(16 + 16 * d, 16)] = g2 * bb[d]
                for k in range(6):
                    pay[e, pl.ds(64 + 16 * k, 16)] = g3 * bb[3 + k]
                return 0
            lax.fori_loop(0, CH, edge_body, 0)

            pltpu.sync_copy(pay, acc.at[dst_i], add=True)
            fire_linear(i + 2, p)

        fire_linear(0, 0)
        fire_linear(1, 1)
        wait_linear(0)
        fire_gathers(0)

        def body2(k, _):
            run_chunk(2 * k, 0)
            run_chunk(2 * k + 1, 1)
            return 0
        lax.fori_loop(0, n_chunks // 2, body2, 0)
        run_chunk(n_chunks - 1, (n_chunks - 1) % 2)

        wait_linear(0)
        wait_gathers(1)
        plsc.subcore_barrier()

        def drain(j, _):
            r0 = base_n + CH * j
            pltpu.sync_copy(acc.at[pl.ds(r0, CH)], pay)
            pltpu.sync_copy(pay, out.at[c, pl.ds(r0, CH)])
            return 0
        lax.fori_loop(0, NPT // CH, drain, 0)
        if rem:
            r0 = base_n + (NPT // CH) * CH
            pltpu.sync_copy(acc.at[pl.ds(r0, rem)], pay.at[pl.ds(0, rem)])
            pltpu.sync_copy(pay.at[pl.ds(0, rem)], out.at[c, pl.ds(r0, rem)])

    return sc_fn(edge_index, WB, U2, V2)


def _finish_kernel(g0_ref, g1h_ref, lng_ref, lnb_ref, ws0_ref, bs0_ref,
                   ws1_ref, bs1_ref, wt0_ref, wt1_ref, wt2_ref, *o_refs):
    gh0 = g0_ref[...]
    gh1 = g1h_ref[...]

    def comp(k):
        return jnp.concatenate(
            [gh0[:, 16 * k:16 * k + 16], gh1[:, 16 * k:16 * k + 16]], axis=1)

    G1 = comp(0)
    G2 = [comp(1), comp(2), comp(3)]
    P = [comp(4 + i) for i in range(6)]

    trP = P[0] + P[1] + P[2]
    Pn2 = (P[0] * P[0] + P[1] * P[1] + P[2] * P[2]
           + 2.0 * (P[3] * P[3] + P[4] * P[4] + P[5] * P[5]))
    nrm = (3.0 * G1 * G1 + 2.0 * (G2[0] * G2[0] + G2[1] * G2[1]
                                  + G2[2] * G2[2]) + Pn2 - trP * trP / 3.0)
    mu = jnp.mean(nrm, axis=1, keepdims=True)
    var = jnp.mean((nrm - mu) ** 2, axis=1, keepdims=True)
    nrm = (nrm - mu) / jnp.sqrt(var + 1e-5) * lng_ref[...] + lnb_ref[...]

    def mm_t(x, w):
        return lax.dot_general(x, w, (((1,), (1,)), ((), ())),
                               preferred_element_type=jnp.float32,
                               precision=lax.Precision.HIGHEST)

    h = mm_t(nrm, ws0_ref[...]) + bs0_ref[...]
    h = h * jax.nn.sigmoid(h)
    h = mm_t(h, ws1_ref[...]) + bs1_ref[...]
    h = h * jax.nn.sigmoid(h)
    n0 = h[:, 0:32]
    n1 = h[:, 32:64]
    n2 = h[:, 64:96]

    A0 = mm_t(G1, wt0_ref[...])
    w0 = mm_t(G2[0], wt1_ref[...])
    w1 = mm_t(G2[1], wt1_ref[...])
    w2 = mm_t(G2[2], wt1_ref[...])
    Pp = [mm_t(P[k], wt2_ref[...]) for k in range(6)]
    t3 = (Pp[0] + Pp[1] + Pp[2]) / 3.0

    diag = n0 * A0
    o_refs[0][...] = diag + n2 * (Pp[0] - t3)
    o_refs[1][...] = n2 * Pp[3] - n1 * w2
    o_refs[2][...] = n2 * Pp[5] + n1 * w1
    o_refs[3][...] = n2 * Pp[3] + n1 * w2
    o_refs[4][...] = diag + n2 * (Pp[1] - t3)
    o_refs[5][...] = n2 * Pp[4] - n1 * w0
    o_refs[6][...] = n2 * Pp[5] - n1 * w1
    o_refs[7][...] = n2 * Pp[4] + n1 * w0
    o_refs[8][...] = diag + n2 * (Pp[2] - t3)


def _node_finish(Gh, ln_g, ln_b, Ws0, bs0, Ws1, bs1, Wt0, Wt1, Wt2):
    perm = jnp.asarray([3 * c + k for k in range(3) for c in range(32)],
                       dtype=jnp.int32)
    ws1p = Ws1[perm, :]
    bs1p = bs1[perm]
    grid = (N_NODES // BN,)
    full = lambda s: pl.BlockSpec(s, lambda i: tuple(0 for _ in s))
    outs = pl.pallas_call(
        _finish_kernel,
        grid=grid,
        in_specs=[
            pl.BlockSpec((BN, 160), lambda i: (i, 0)),
            pl.BlockSpec((BN, 160), lambda i: (i, 0)),
            full((1, 32)), full((1, 32)), full((64, 32)), full((1, 64)),
            full((96, 64)), full((1, 96)), full((32, 32)), full((32, 32)),
            full((32, 32)),
        ],
        out_specs=[pl.BlockSpec((BN, 32), lambda i: (i, 0))] * 9,
        out_shape=[jax.ShapeDtypeStruct((N_NODES, 32), jnp.float32)] * 9,
    )(Gh[0], Gh[1], ln_g[None, :], ln_b[None, :], Ws0, bs0[None, :],
      ws1p, bs1p[None, :], Wt0, Wt1, Wt2)
    return jnp.stack(outs, axis=-1).reshape(N_NODES, UNITS, 3, 3)


def kernel(node_type, edge_index, edge_attr, bond_dist, bond_vec, emb_table,
           Wd1, bd1, Wd2, bd2, Wd3, bd3, W_emb2, b_emb2, W_emb3, b_emb3,
           Wt0, Wt1, Wt2, Ws0, bs0, Ws1, bs1, ln_g, ln_b):
    WB, edge_feat = _edge_precompute(
        edge_attr, bond_dist, bond_vec, Wd1, bd1, Wd2, bd2, Wd3, bd3,
        W_emb3, b_emb3)
    U2, V2 = _node_precompute(node_type, emb_table, W_emb2, b_emb2)
    Gh = _sc_scatter(edge_index, WB, U2, V2)
    X = _node_finish(Gh, ln_g, ln_b, Ws0, bs0, Ws1, bs1, Wt0, Wt1, Wt2)
    return X, edge_feat


# speedup vs baseline: 73.3832x; 1.1590x over previous
import functools

import jax
import jax.numpy as jnp
from jax import lax
from jax.experimental import pallas as pl
from jax.experimental.pallas import tpu as pltpu
from jax.experimental.pallas import tpu_sc as plsc

N_NODES = 10000
N_EDGES = 160000
UNITS = 32
CUTOFF = 5.0

NC = 2
NS = 16
CH = 40
EPW = N_EDGES // NS
NPT = N_NODES // NS
BE = 3200
BN = 1000


def _edge_kernel(ea_ref, bd_ref, bv_ref, wc_ref, bc_ref, we3_ref, be3_ref,
                 wb_ref, ef_ref):
    ea = ea_ref[...]

    def mm(x, w):
        return lax.dot_general(x, w, (((1,), (1,)), ((), ())),
                               preferred_element_type=jnp.float32,
                               precision=lax.Precision.HIGHEST)

    wb_ref[...] = mm(ea, wc_ref[...]) + bc_ref[...]

    r = bd_ref[...]
    c = jnp.where(r <= CUTOFF, 0.5 * (jnp.cos(jnp.pi * r / CUTOFF) + 1.0), 0.0)

    v = bv_ref[...]
    inv = 1.0 / jnp.sqrt(jnp.sum(v * v, axis=0, keepdims=True))
    ev = v * inv
    sq = ev * ev
    evr = jnp.concatenate([ev[1:], ev[:1]], axis=0)
    cr = ev * evr
    comp = jnp.concatenate([ev, sq, cr, c, ev, ev], axis=0)
    basc = comp.T

    wb_ref[:, 48:64] = basc
    wb_ref[:, 112:128] = basc

    ef_ref[...] = mm(ea, we3_ref[...]) + be3_ref[...]


def _edge_precompute(edge_attr, bond_dist, bond_vec, Wd1, bd1, Wd2, bd2,
                     Wd3, bd3, W_emb3, b_emb3):
    z16 = jnp.zeros((16, 32), jnp.float32)
    wcomb = jnp.concatenate(
        [Wd1[:16], Wd2[:16], Wd3[:16], z16,
         Wd1[16:], Wd2[16:], Wd3[16:], z16], axis=0)
    zb = jnp.zeros((16,), jnp.float32)
    bcomb = jnp.concatenate(
        [bd1[:16], bd2[:16], bd3[:16], zb,
         bd1[16:], bd2[16:], bd3[16:], zb], axis=0)
    grid = (N_EDGES // BE,)
    return pl.pallas_call(
        _edge_kernel,
        grid=grid,
        in_specs=[
            pl.BlockSpec((BE, 32), lambda i: (i, 0)),
            pl.BlockSpec((1, BE), lambda i: (0, i)),
            pl.BlockSpec((3, BE), lambda i: (0, i)),
            pl.BlockSpec((128, 32), lambda i: (0, 0)),
            pl.BlockSpec((1, 128), lambda i: (0, 0)),
            pl.BlockSpec((32, 32), lambda i: (0, 0)),
            pl.BlockSpec((1, 32), lambda i: (0, 0)),
        ],
        out_specs=[
            pl.BlockSpec((BE, 128), lambda i: (i, 0)),
            pl.BlockSpec((BE, 32), lambda i: (i, 0)),
        ],
        out_shape=[
            jax.ShapeDtypeStruct((N_EDGES, 128), jnp.float32),
            jax.ShapeDtypeStruct((N_EDGES, 32), jnp.float32),
        ],
    )(edge_attr, bond_dist[None, :], bond_vec.T, wcomb, bcomb[None, :],
      W_emb3, b_emb3[None, :])


def _node_kernel(nt_ref, emb_ref, wa_ref, wb_ref, b2_ref, u_ref, v_ref):
    nt = nt_ref[...]
    iota = lax.broadcasted_iota(jnp.int32, (BN, 128), 1)
    oh = (nt == iota).astype(jnp.float32)

    def mm_t(x, w):
        return lax.dot_general(x, w, (((1,), (1,)), ((), ())),
                               preferred_element_type=jnp.float32,
                               precision=lax.Precision.HIGHEST)

    def mm(x, w):
        return lax.dot_general(x, w, (((1,), (0,)), ((), ())),
                               preferred_element_type=jnp.float32,
                               precision=lax.Precision.HIGHEST)

    ma = mm_t(emb_ref[...], wa_ref[...])
    mb = mm_t(emb_ref[...], wb_ref[...])
    u = mm(oh, ma)
    v = mm(oh, mb) + b2_ref[...]
    u_ref[0] = u[:, :16]
    u_ref[1] = u[:, 16:]
    v_ref[0] = v[:, :16]
    v_ref[1] = v[:, 16:]


def _node_precompute(node_type, emb_table, W_emb2, b_emb2):
    emb_pad = jnp.zeros((128, 32), jnp.float32).at[:emb_table.shape[0]].set(
        emb_table)
    wa = W_emb2[:, :UNITS]
    wb = W_emb2[:, UNITS:]
    grid = (N_NODES // BN,)
    full = lambda s: pl.BlockSpec(s, lambda i: tuple(0 for _ in s))
    return pl.pallas_call(
        _node_kernel,
        grid=grid,
        in_specs=[
            pl.BlockSpec((BN, 1), lambda i: (i, 0)),
            full((128, 32)), full((32, 32)), full((32, 32)), full((1, 32)),
        ],
        out_specs=[
            pl.BlockSpec((2, BN, 16), lambda i: (0, i, 0)),
            pl.BlockSpec((2, BN, 16), lambda i: (0, i, 0)),
        ],
        out_shape=[
            jax.ShapeDtypeStruct((2, N_NODES, 16), jnp.float32),
            jax.ShapeDtypeStruct((2, N_NODES, 16), jnp.float32),
        ],
    )(node_type[:, None], emb_pad, wa, wb, b_emb2[None, :])


def _sc_scatter(edge_index, WB, U2, V2):
    mesh = plsc.VectorSubcoreMesh(core_axis_name="c", subcore_axis_name="s")
    n_chunks = EPW // CH

    vset = lambda: [pltpu.VMEM((CH,), jnp.int32),
                    pltpu.VMEM((CH,), jnp.int32),
                    pltpu.VMEM((CH, 64), jnp.float32),
                    pltpu.VMEM((CH, 16), jnp.float32),
                    pltpu.VMEM((CH, 16), jnp.float32)]

    @functools.partial(
        pl.kernel,
        out_type=jax.ShapeDtypeStruct((NC, N_NODES, 160), jnp.float32),
        mesh=mesh,
        compiler_params=pltpu.CompilerParams(use_tc_tiling_on_sc=False),
        scratch_types=[
            pltpu.VMEM_SHARED((N_NODES, 160), jnp.float32),
            vset(), vset(),
            pltpu.VMEM((CH, 160), jnp.float32),
            pltpu.VMEM((CH, 160), jnp.float32),
            pltpu.VMEM((CH,), jnp.int32),
            pltpu.VMEM((CH,), jnp.int32),
            pltpu.SemaphoreType.DMA, pltpu.SemaphoreType.DMA,
            pltpu.SemaphoreType.DMA, pltpu.SemaphoreType.DMA,
            pltpu.SemaphoreType.DMA, pltpu.SemaphoreType.DMA,
            pltpu.SemaphoreType.DMA, pltpu.SemaphoreType.DMA,
        ],
    )
    def sc_fn(ei, wb, u2, v2, out, acc, set0, set1, pay0, pay1, dsts0, dsts1,
              sl0, sl1, sg0, sg1, ss0, ss1, sd0, sd1):
        c = lax.axis_index("c")
        s = lax.axis_index("s")
        base_n = s * NPT
        sets = (set0, set1)
        pays = (pay0, pay1)
        dstss = (dsts0, dsts1)
        sem_l = (sl0, sl1)
        sem_g = (sg0, sg1)
        sem_s = (ss0, ss1)
        sem_d = (sd0, sd1)

        def zrow(i, _):
            for k in range(10):
                pay0[i, pl.ds(16 * k, 16)] = jnp.zeros((16,), jnp.float32)
            return 0
        lax.fori_loop(0, CH, zrow, 0)

        def zcopy(j, _):
            pltpu.sync_copy(pay0, acc.at[pl.ds(base_n + CH * j, CH)])
            return 0
        lax.fori_loop(0, NPT // CH, zcopy, 0)
        rem = NPT - (NPT // CH) * CH
        if rem:
            pltpu.sync_copy(pay0.at[pl.ds(0, rem)],
                            acc.at[pl.ds(base_n + (NPT // CH) * CH, rem)])
        plsc.subcore_barrier()

        def e_of(i):
            return s * EPW + jnp.minimum(i, n_chunks - 1) * CH

        def fire_linear(i, p):
            src_i, dst_i, wc, _, _ = sets[p]
            e0 = e_of(i)
            pltpu.async_copy(ei.at[0, pl.ds(e0, CH)], src_i, sem_l[p])
            pltpu.async_copy(ei.at[1, pl.ds(e0, CH)], dst_i, sem_l[p])
            pltpu.async_copy(wb.at[pl.ds(e0, CH), pl.ds(64 * c, 64)],
                             wc, sem_l[p])

        def wait_linear(p):
            src_i, dst_i, wc, _, _ = sets[p]
            e0 = s * EPW
            pltpu.make_async_copy(ei.at[0, pl.ds(e0, CH)], src_i, sem_l[p]).wait()
            pltpu.make_async_copy(ei.at[1, pl.ds(e0, CH)], dst_i, sem_l[p]).wait()
            pltpu.make_async_copy(wb.at[pl.ds(e0, CH), pl.ds(0, 64)],
                                  wc, sem_l[p]).wait()

        def fire_gathers(p):
            src_i, dst_i, _, ur, vr = sets[p]
            pltpu.async_copy(u2.at[c].at[src_i], ur, sem_g[p])
            pltpu.async_copy(v2.at[c].at[dst_i], vr, sem_g[p])

        def wait_gathers(p):
            src_i, dst_i, _, ur, vr = sets[p]
            pltpu.make_async_copy(u2.at[c].at[src_i], ur, sem_g[p]).wait()
            pltpu.make_async_copy(v2.at[c].at[dst_i], vr, sem_g[p]).wait()

        def fire_scatter(p):
            pltpu.async_copy(pays[p], acc.at[dstss[p]], sem_s[p], add=True)

        def wait_scatter(p):
            pltpu.make_async_copy(pays[p], acc.at[dstss[p]], sem_s[p]).wait()

        def run_chunk(i, p, first):
            q = 1 - p
            src_i, dst_i, wc, ur, vr = sets[p]
            pay = pays[p]
            wait_linear(q)
            fire_gathers(q)
            wait_gathers(p)
            if not first:
                wait_scatter(p)
            e0 = s * EPW + i * CH
            pltpu.async_copy(ei.at[1, pl.ds(e0, CH)], dstss[p], sem_d[p])

            def edge_body(e, _):
                bb = wc[e, pl.ds(48, 16)]
                zc = (ur[e, :] + vr[e, :]) * bb[9]
                g1 = zc * wc[e, pl.ds(0, 16)]
                g2 = zc * wc[e, pl.ds(16, 16)]
                g3 = zc * wc[e, pl.ds(32, 16)]
                pay[e, pl.ds(0, 16)] = g1
                for d in range(3):
                    pay[e, pl.ds(16 + 16 * d, 16)] = g2 * bb[d]
                for k in range(6):
                    pay[e, pl.ds(64 + 16 * k, 16)] = g3 * bb[3 + k]
                return 0
            lax.fori_loop(0, CH, edge_body, 0)

            pltpu.make_async_copy(ei.at[1, pl.ds(e0, CH)], dstss[p],
                                  sem_d[p]).wait()
            fire_scatter(p)
            fire_linear(i + 2, p)

        fire_linear(0, 0)
        fire_linear(1, 1)
        wait_linear(0)
        fire_gathers(0)

        run_chunk(0, 0, True)
        run_chunk(1, 1, True)

        def body2(k, _):
            run_chunk(2 * k, 0, False)
            run_chunk(2 * k + 1, 1, False)
            return 0
        lax.fori_loop(1, n_chunks // 2, body2, 0)

        wait_linear(1)
        wait_gathers(0)
        wait_scatter(0)
        wait_scatter(1)
        plsc.subcore_barrier()

        def drain(j, _):
            r0 = base_n + CH * j
            pltpu.sync_copy(acc.at[pl.ds(r0, CH)], pay0)
            pltpu.sync_copy(pay0, out.at[c, pl.ds(r0, CH)])
            return 0
        lax.fori_loop(0, NPT // CH, drain, 0)
        if rem:
            r0 = base_n + (NPT // CH) * CH
            pltpu.sync_copy(acc.at[pl.ds(r0, rem)], pay0.at[pl.ds(0, rem)])
            pltpu.sync_copy(pay0.at[pl.ds(0, rem)], out.at[c, pl.ds(r0, rem)])

    return sc_fn(edge_index, WB, U2, V2)


def _finish_kernel(g0_ref, g1h_ref, lng_ref, lnb_ref, ws0_ref, bs0_ref,
                   ws1_ref, bs1_ref, wt0_ref, wt1_ref, wt2_ref, *o_refs):
    gh0 = g0_ref[...]
    gh1 = g1h_ref[...]

    def comp(k):
        return jnp.concatenate(
            [gh0[:, 16 * k:16 * k + 16], gh1[:, 16 * k:16 * k + 16]], axis=1)

    G1 = comp(0)
    G2 = [comp(1), comp(2), comp(3)]
    P = [comp(4 + i) for i in range(6)]

    trP = P[0] + P[1] + P[2]
    Pn2 = (P[0] * P[0] + P[1] * P[1] + P[2] * P[2]
           + 2.0 * (P[3] * P[3] + P[4] * P[4] + P[5] * P[5]))
    nrm = (3.0 * G1 * G1 + 2.0 * (G2[0] * G2[0] + G2[1] * G2[1]
                                  + G2[2] * G2[2]) + Pn2 - trP * trP / 3.0)
    mu = jnp.mean(nrm, axis=1, keepdims=True)
    var = jnp.mean((nrm - mu) ** 2, axis=1, keepdims=True)
    nrm = (nrm - mu) / jnp.sqrt(var + 1e-5) * lng_ref[...] + lnb_ref[...]

    def mm_t(x, w):
        return lax.dot_general(x, w, (((1,), (1,)), ((), ())),
                               preferred_element_type=jnp.float32,
                               precision=lax.Precision.HIGHEST)

    h = mm_t(nrm, ws0_ref[...]) + bs0_ref[...]
    h = h * jax.nn.sigmoid(h)
    h = mm_t(h, ws1_ref[...]) + bs1_ref[...]
    h = h * jax.nn.sigmoid(h)
    n0 = h[:, 0:32]
    n1 = h[:, 32:64]
    n2 = h[:, 64:96]

    A0 = mm_t(G1, wt0_ref[...])
    w0 = mm_t(G2[0], wt1_ref[...])
    w1 = mm_t(G2[1], wt1_ref[...])
    w2 = mm_t(G2[2], wt1_ref[...])
    Pp = [mm_t(P[k], wt2_ref[...]) for k in range(6)]
    t3 = (Pp[0] + Pp[1] + Pp[2]) / 3.0

    diag = n0 * A0
    o_refs[0][...] = diag + n2 * (Pp[0] - t3)
    o_refs[1][...] = n2 * Pp[3] - n1 * w2
    o_refs[2][...] = n2 * Pp[5] + n1 * w1
    o_refs[3][...] = n2 * Pp[3] + n1 * w2
    o_refs[4][...] = diag + n2 * (Pp[1] - t3)
    o_refs[5][...] = n2 * Pp[4] - n1 * w0
    o_refs[6][...] = n2 * Pp[5] - n1 * w1
    o_refs[7][...] = n2 * Pp[4] + n1 * w0
    o_refs[8][...] = diag + n2 * (Pp[2] - t3)


def _node_finish(Gh, ln_g, ln_b, Ws0, bs0, Ws1, bs1, Wt0, Wt1, Wt2):
    perm = jnp.asarray([3 * c + k for k in range(3) for c in range(32)],
                       dtype=jnp.int32)
    ws1p = Ws1[perm, :]
    bs1p = bs1[perm]
    grid = (N_NODES // BN,)
    full = lambda s: pl.BlockSpec(s, lambda i: tuple(0 for _ in s))
    outs = pl.pallas_call(
        _finish_kernel,
        grid=grid,
        in_specs=[
            pl.BlockSpec((BN, 160), lambda i: (i, 0)),
            pl.BlockSpec((BN, 160), lambda i: (i, 0)),
            full((1, 32)), full((1, 32)), full((64, 32)), full((1, 64)),
            full((96, 64)), full((1, 96)), full((32, 32)), full((32, 32)),
            full((32, 32)),
        ],
        out_specs=[pl.BlockSpec((BN, 32), lambda i: (i, 0))] * 9,
        out_shape=[jax.ShapeDtypeStruct((N_NODES, 32), jnp.float32)] * 9,
    )(Gh[0], Gh[1], ln_g[None, :], ln_b[None, :], Ws0, bs0[None, :],
      ws1p, bs1p[None, :], Wt0, Wt1, Wt2)
    return jnp.stack(outs, axis=-1).reshape(N_NODES, UNITS, 3, 3)


def kernel(node_type, edge_index, edge_attr, bond_dist, bond_vec, emb_table,
           Wd1, bd1, Wd2, bd2, Wd3, bd3, W_emb2, b_emb2, W_emb3, b_emb3,
           Wt0, Wt1, Wt2, Ws0, bs0, Ws1, bs1, ln_g, ln_b):
    WB, edge_feat = _edge_precompute(
        edge_attr, bond_dist, bond_vec, Wd1, bd1, Wd2, bd2, Wd3, bd3,
        W_emb3, b_emb3)
    U2, V2 = _node_precompute(node_type, emb_table, W_emb2, b_emb2)
    Gh = _sc_scatter(edge_index, WB, U2, V2)
    X = _node_finish(Gh, ln_g, ln_b, Ws0, bs0, Ws1, bs1, Wt0, Wt1, Wt2)
    return X, edge_feat


# speedup vs baseline: 75.9950x; 1.0356x over previous
import functools

import jax
import jax.numpy as jnp
from jax import lax
from jax.experimental import pallas as pl
from jax.experimental.pallas import tpu as pltpu
from jax.experimental.pallas import tpu_sc as plsc

N_NODES = 10000
N_EDGES = 160000
UNITS = 32
CUTOFF = 5.0

NC = 2
NS = 16
CH = 40
EPW = N_EDGES // NS
NPT = N_NODES // NS
BE = 3200
BN = 1000


def _edge_kernel(ea_ref, bd_ref, bv_ref, wc_ref, bc_ref, we3_ref, be3_ref,
                 wb_ref, ef_ref):
    ea = ea_ref[...]

    def mm(x, w):
        return lax.dot_general(x, w, (((1,), (1,)), ((), ())),
                               preferred_element_type=jnp.float32,
                               precision=lax.Precision.HIGHEST)

    wb_ref[...] = mm(ea, wc_ref[...]) + bc_ref[...]

    r = bd_ref[...]
    c = jnp.where(r <= CUTOFF, 0.5 * (jnp.cos(jnp.pi * r / CUTOFF) + 1.0), 0.0)

    v = bv_ref[...]
    inv = 1.0 / jnp.sqrt(jnp.sum(v * v, axis=0, keepdims=True))
    ev = v * inv
    sq = ev * ev
    evr = jnp.concatenate([ev[1:], ev[:1]], axis=0)
    cr = ev * evr
    comp = jnp.concatenate([ev, sq, cr, c, ev, ev], axis=0)
    basc = comp.T

    wb_ref[:, 48:64] = basc
    wb_ref[:, 112:128] = basc

    ef_ref[...] = mm(ea, we3_ref[...]) + be3_ref[...]


def _edge_precompute(edge_attr, bond_dist, bond_vec, Wd1, bd1, Wd2, bd2,
                     Wd3, bd3, W_emb3, b_emb3):
    z16 = jnp.zeros((16, 32), jnp.float32)
    wcomb = jnp.concatenate(
        [Wd1[:16], Wd2[:16], Wd3[:16], z16,
         Wd1[16:], Wd2[16:], Wd3[16:], z16], axis=0)
    zb = jnp.zeros((16,), jnp.float32)
    bcomb = jnp.concatenate(
        [bd1[:16], bd2[:16], bd3[:16], zb,
         bd1[16:], bd2[16:], bd3[16:], zb], axis=0)
    grid = (N_EDGES // BE,)
    return pl.pallas_call(
        _edge_kernel,
        grid=grid,
        in_specs=[
            pl.BlockSpec((BE, 32), lambda i: (i, 0)),
            pl.BlockSpec((1, BE), lambda i: (0, i)),
            pl.BlockSpec((3, BE), lambda i: (0, i)),
            pl.BlockSpec((128, 32), lambda i: (0, 0)),
            pl.BlockSpec((1, 128), lambda i: (0, 0)),
            pl.BlockSpec((32, 32), lambda i: (0, 0)),
            pl.BlockSpec((1, 32), lambda i: (0, 0)),
        ],
        out_specs=[
            pl.BlockSpec((BE, 128), lambda i: (i, 0)),
            pl.BlockSpec((BE, 32), lambda i: (i, 0)),
        ],
        out_shape=[
            jax.ShapeDtypeStruct((N_EDGES, 128), jnp.float32),
            jax.ShapeDtypeStruct((N_EDGES, 32), jnp.float32),
        ],
    )(edge_attr, bond_dist[None, :], bond_vec.T, wcomb, bcomb[None, :],
      W_emb3, b_emb3[None, :])


def _node_kernel(nt_ref, emb_ref, wa_ref, wb_ref, b2_ref, u_ref, v_ref):
    nt = nt_ref[...]
    iota = lax.broadcasted_iota(jnp.int32, (BN, 128), 1)
    oh = (nt == iota).astype(jnp.float32)

    def mm_t(x, w):
        return lax.dot_general(x, w, (((1,), (1,)), ((), ())),
                               preferred_element_type=jnp.float32,
                               precision=lax.Precision.HIGHEST)

    def mm(x, w):
        return lax.dot_general(x, w, (((1,), (0,)), ((), ())),
                               preferred_element_type=jnp.float32,
                               precision=lax.Precision.HIGHEST)

    ma = mm_t(emb_ref[...], wa_ref[...])
    mb = mm_t(emb_ref[...], wb_ref[...])
    u = mm(oh, ma)
    v = mm(oh, mb) + b2_ref[...]
    u_ref[0] = u[:, :16]
    u_ref[1] = u[:, 16:]
    v_ref[0] = v[:, :16]
    v_ref[1] = v[:, 16:]


def _node_precompute(node_type, emb_table, W_emb2, b_emb2):
    emb_pad = jnp.zeros((128, 32), jnp.float32).at[:emb_table.shape[0]].set(
        emb_table)
    wa = W_emb2[:, :UNITS]
    wb = W_emb2[:, UNITS:]
    grid = (N_NODES // BN,)
    full = lambda s: pl.BlockSpec(s, lambda i: tuple(0 for _ in s))
    return pl.pallas_call(
        _node_kernel,
        grid=grid,
        in_specs=[
            pl.BlockSpec((BN, 1), lambda i: (i, 0)),
            full((128, 32)), full((32, 32)), full((32, 32)), full((1, 32)),
        ],
        out_specs=[
            pl.BlockSpec((2, BN, 16), lambda i: (0, i, 0)),
            pl.BlockSpec((2, BN, 16), lambda i: (0, i, 0)),
        ],
        out_shape=[
            jax.ShapeDtypeStruct((2, N_NODES, 16), jnp.float32),
            jax.ShapeDtypeStruct((2, N_NODES, 16), jnp.float32),
        ],
    )(node_type[:, None], emb_pad, wa, wb, b_emb2[None, :])


def _sc_scatter(edge_index, WB, U2, V2):
    mesh = plsc.VectorSubcoreMesh(core_axis_name="c", subcore_axis_name="s")
    n_chunks = EPW // CH

    vset = lambda: [pltpu.VMEM((CH,), jnp.int32),
                    pltpu.VMEM((CH,), jnp.int32),
                    pltpu.VMEM((CH, 64), jnp.float32),
                    pltpu.VMEM((CH, 16), jnp.float32),
                    pltpu.VMEM((CH, 16), jnp.float32)]

    @functools.partial(
        pl.kernel,
        out_type=[jax.ShapeDtypeStruct((NC, N_NODES, 128), jnp.float32),
                  jax.ShapeDtypeStruct((NC, N_NODES, 32), jnp.float32)],
        mesh=mesh,
        compiler_params=pltpu.CompilerParams(use_tc_tiling_on_sc=False),
        scratch_types=[
            pltpu.VMEM_SHARED((N_NODES, 160), jnp.float32),
            vset(), vset(),
            pltpu.VMEM((CH, 160), jnp.float32),
            pltpu.VMEM((CH, 160), jnp.float32),
            pltpu.VMEM((CH,), jnp.int32),
            pltpu.VMEM((CH,), jnp.int32),
            pltpu.SemaphoreType.DMA, pltpu.SemaphoreType.DMA,
            pltpu.SemaphoreType.DMA, pltpu.SemaphoreType.DMA,
            pltpu.SemaphoreType.DMA, pltpu.SemaphoreType.DMA,
            pltpu.SemaphoreType.DMA, pltpu.SemaphoreType.DMA,
        ],
    )
    def sc_fn(ei, wb, u2, v2, out, out2, acc, set0, set1, pay0, pay1,
              dsts0, dsts1, sl0, sl1, sg0, sg1, ss0, ss1, sd0, sd1):
        c = lax.axis_index("c")
        s = lax.axis_index("s")
        base_n = s * NPT
        sets = (set0, set1)
        pays = (pay0, pay1)
        dstss = (dsts0, dsts1)
        sem_l = (sl0, sl1)
        sem_g = (sg0, sg1)
        sem_s = (ss0, ss1)
        sem_d = (sd0, sd1)

        def zrow(i, _):
            for k in range(10):
                pay0[i, pl.ds(16 * k, 16)] = jnp.zeros((16,), jnp.float32)
            return 0
        lax.fori_loop(0, CH, zrow, 0)

        def zcopy(j, _):
            pltpu.sync_copy(pay0, acc.at[pl.ds(base_n + CH * j, CH)])
            return 0
        lax.fori_loop(0, NPT // CH, zcopy, 0)
        rem = NPT - (NPT // CH) * CH
        if rem:
            pltpu.sync_copy(pay0.at[pl.ds(0, rem)],
                            acc.at[pl.ds(base_n + (NPT // CH) * CH, rem)])
        plsc.subcore_barrier()

        def e_of(i):
            return s * EPW + jnp.minimum(i, n_chunks - 1) * CH

        def fire_linear(i, p):
            src_i, dst_i, wc, _, _ = sets[p]
            e0 = e_of(i)
            pltpu.async_copy(ei.at[0, pl.ds(e0, CH)], src_i, sem_l[p])
            pltpu.async_copy(ei.at[1, pl.ds(e0, CH)], dst_i, sem_l[p])
            pltpu.async_copy(wb.at[pl.ds(e0, CH), pl.ds(64 * c, 64)],
                             wc, sem_l[p])

        def wait_linear(p):
            src_i, dst_i, wc, _, _ = sets[p]
            e0 = s * EPW
            pltpu.make_async_copy(ei.at[0, pl.ds(e0, CH)], src_i, sem_l[p]).wait()
            pltpu.make_async_copy(ei.at[1, pl.ds(e0, CH)], dst_i, sem_l[p]).wait()
            pltpu.make_async_copy(wb.at[pl.ds(e0, CH), pl.ds(0, 64)],
                                  wc, sem_l[p]).wait()

        def fire_gathers(p):
            src_i, dst_i, _, ur, vr = sets[p]
            pltpu.async_copy(u2.at[c].at[src_i], ur, sem_g[p])
            pltpu.async_copy(v2.at[c].at[dst_i], vr, sem_g[p])

        def wait_gathers(p):
            src_i, dst_i, _, ur, vr = sets[p]
            pltpu.make_async_copy(u2.at[c].at[src_i], ur, sem_g[p]).wait()
            pltpu.make_async_copy(v2.at[c].at[dst_i], vr, sem_g[p]).wait()

        def fire_scatter(p):
            pltpu.async_copy(pays[p], acc.at[dstss[p]], sem_s[p], add=True)

        def wait_scatter(p):
            pltpu.make_async_copy(pays[p], acc.at[dstss[p]], sem_s[p]).wait()

        def run_chunk(i, p, first):
            q = 1 - p
            src_i, dst_i, wc, ur, vr = sets[p]
            pay = pays[p]
            wait_linear(q)
            fire_gathers(q)
            wait_gathers(p)
            if not first:
                wait_scatter(p)
            e0 = s * EPW + i * CH
            pltpu.async_copy(ei.at[1, pl.ds(e0, CH)], dstss[p], sem_d[p])

            def edge_body(e, _):
                bb = wc[e, pl.ds(48, 16)]
                zc = (ur[e, :] + vr[e, :]) * bb[9]
                g1 = zc * wc[e, pl.ds(0, 16)]
                g2 = zc * wc[e, pl.ds(16, 16)]
                g3 = zc * wc[e, pl.ds(32, 16)]
                pay[e, pl.ds(0, 16)] = g1
                for d in range(3):
                    pay[e, pl.ds(16 + 16 * d, 16)] = g2 * bb[d]
                for k in range(6):
                    pay[e, pl.ds(64 + 16 * k, 16)] = g3 * bb[3 + k]
                return 0
            lax.fori_loop(0, CH, edge_body, 0)

            pltpu.make_async_copy(ei.at[1, pl.ds(e0, CH)], dstss[p],
                                  sem_d[p]).wait()
            fire_scatter(p)
            fire_linear(i + 2, p)

        fire_linear(0, 0)
        fire_linear(1, 1)
        wait_linear(0)
        fire_gathers(0)

        run_chunk(0, 0, True)
        run_chunk(1, 1, True)

        def body2(k, _):
            run_chunk(2 * k, 0, False)
            run_chunk(2 * k + 1, 1, False)
            return 0
        lax.fori_loop(1, n_chunks // 2, body2, 0)

        wait_linear(1)
        wait_gathers(0)
        wait_scatter(0)
        wait_scatter(1)
        plsc.subcore_barrier()

        def drain(j, _):
            r0 = base_n + CH * j
            pltpu.sync_copy(acc.at[pl.ds(r0, CH)], pay0)
            pltpu.sync_copy(pay0.at[pl.ds(0, CH), pl.ds(0, 128)],
                            out.at[c, pl.ds(r0, CH)])
            pltpu.sync_copy(pay0.at[pl.ds(0, CH), pl.ds(128, 32)],
                            out2.at[c, pl.ds(r0, CH)])
            return 0
        lax.fori_loop(0, NPT // CH, drain, 0)
        if rem:
            r0 = base_n + (NPT // CH) * CH
            pltpu.sync_copy(acc.at[pl.ds(r0, rem)], pay0.at[pl.ds(0, rem)])
            pltpu.sync_copy(pay0.at[pl.ds(0, rem), pl.ds(0, 128)],
                            out.at[c, pl.ds(r0, rem)])
            pltpu.sync_copy(pay0.at[pl.ds(0, rem), pl.ds(128, 32)],
                            out2.at[c, pl.ds(r0, rem)])

    return sc_fn(edge_index, WB, U2, V2)


def _finish_kernel(ga0_ref, ga1_ref, gb0_ref, gb1_ref, lng_ref, lnb_ref,
                   ws0_ref, bs0_ref, ws1_ref, bs1_ref,
                   wa0_ref, wa1_ref, wb0_ref, wb1_ref, *o_refs):
    ga = (ga0_ref[...], ga1_ref[...])
    gb = (gb0_ref[...], gb1_ref[...])

    def half_norm(a, b):
        q = a * a
        s = lambda k: q[:, 16 * k:16 * k + 16]
        qb = b * b
        trp = a[:, 64:80] + a[:, 80:96] + a[:, 96:112]
        return (3.0 * s(0) + 2.0 * (s(1) + s(2) + s(3))
                + s(4) + s(5) + s(6)
                + 2.0 * (s(7) + qb[:, 0:16] + qb[:, 16:32])
                - trp * trp / 3.0)

    nrm = jnp.concatenate([half_norm(ga[0], gb[0]),
                           half_norm(ga[1], gb[1])], axis=1)
    mu = jnp.mean(nrm, axis=1, keepdims=True)
    var = jnp.mean((nrm - mu) ** 2, axis=1, keepdims=True)
    nrm = (nrm - mu) / jnp.sqrt(var + 1e-5) * lng_ref[...] + lnb_ref[...]

    def mm_t(x, w):
        return lax.dot_general(x, w, (((1,), (1,)), ((), ())),
                               preferred_element_type=jnp.float32,
                               precision=lax.Precision.HIGHEST)

    h = mm_t(nrm, ws0_ref[...]) + bs0_ref[...]
    h = h * jax.nn.sigmoid(h)
    h = mm_t(h, ws1_ref[...]) + bs1_ref[...]
    h = h * jax.nn.sigmoid(h)
    n0 = h[:, 0:32]
    n1 = h[:, 32:64]
    n2 = h[:, 64:96]

    pb = (mm_t(ga[0], wa0_ref[...]) + mm_t(ga[1], wa1_ref[...]))
    pb2 = (mm_t(gb[0], wb0_ref[...]) + mm_t(gb[1], wb1_ref[...]))
    A0 = pb[:, 0:32]
    w0 = pb[:, 32:64]
    w1 = pb[:, 64:96]
    w2 = pb[:, 96:128]
    Pp = [pb[:, 128:160], pb[:, 160:192], pb[:, 192:224], pb[:, 224:256],
          pb2[:, 0:32], pb2[:, 32:64]]
    t3 = (Pp[0] + Pp[1] + Pp[2]) / 3.0

    diag = n0 * A0
    o_refs[0][...] = diag + n2 * (Pp[0] - t3)
    o_refs[1][...] = n2 * Pp[3] - n1 * w2
    o_refs[2][...] = n2 * Pp[5] + n1 * w1
    o_refs[3][...] = n2 * Pp[3] + n1 * w2
    o_refs[4][...] = diag + n2 * (Pp[1] - t3)
    o_refs[5][...] = n2 * Pp[4] - n1 * w0
    o_refs[6][...] = n2 * Pp[5] - n1 * w1
    o_refs[7][...] = n2 * Pp[4] + n1 * w0
    o_refs[8][...] = diag + n2 * (Pp[2] - t3)


def _node_finish(GhA, GhB, ln_g, ln_b, Ws0, bs0, Ws1, bs1, Wt0, Wt1, Wt2):
    perm = jnp.asarray([3 * c + k for k in range(3) for c in range(32)],
                       dtype=jnp.int32)
    ws1p = Ws1[perm, :]
    bs1p = bs1[perm]
    comps_a = [Wt0, Wt1, Wt1, Wt1, Wt2, Wt2, Wt2, Wt2]
    wa = [jnp.zeros((256, 128), jnp.float32) for _ in range(2)]
    wb = [jnp.zeros((64, 32), jnp.float32) for _ in range(2)]
    for h in range(2):
        for k, Wk in enumerate(comps_a):
            wa[h] = wa[h].at[32 * k:32 * k + 32, 16 * k:16 * k + 16].set(
                Wk[:, 16 * h:16 * h + 16])
        for k in range(2):
            wb[h] = wb[h].at[32 * k:32 * k + 32, 16 * k:16 * k + 16].set(
                Wt2[:, 16 * h:16 * h + 16])
    grid = (N_NODES // BN,)
    full = lambda s: pl.BlockSpec(s, lambda i: tuple(0 for _ in s))
    outs = pl.pallas_call(
        _finish_kernel,
        grid=grid,
        in_specs=[
            pl.BlockSpec((BN, 128), lambda i: (i, 0)),
            pl.BlockSpec((BN, 128), lambda i: (i, 0)),
            pl.BlockSpec((BN, 32), lambda i: (i, 0)),
            pl.BlockSpec((BN, 32), lambda i: (i, 0)),
            full((1, 32)), full((1, 32)), full((64, 32)), full((1, 64)),
            full((96, 64)), full((1, 96)),
            full((256, 128)), full((256, 128)), full((64, 32)), full((64, 32)),
        ],
        out_specs=[pl.BlockSpec((BN, 32), lambda i: (i, 0))] * 9,
        out_shape=[jax.ShapeDtypeStruct((N_NODES, 32), jnp.float32)] * 9,
    )(GhA[0], GhA[1], GhB[0], GhB[1], ln_g[None, :], ln_b[None, :],
      Ws0, bs0[None, :], ws1p, bs1p[None, :], wa[0], wa[1], wb[0], wb[1])
    return jnp.stack(outs, axis=-1).reshape(N_NODES, UNITS, 3, 3)


def kernel(node_type, edge_index, edge_attr, bond_dist, bond_vec, emb_table,
           Wd1, bd1, Wd2, bd2, Wd3, bd3, W_emb2, b_emb2, W_emb3, b_emb3,
           Wt0, Wt1, Wt2, Ws0, bs0, Ws1, bs1, ln_g, ln_b):
    WB, edge_feat = _edge_precompute(
        edge_attr, bond_dist, bond_vec, Wd1, bd1, Wd2, bd2, Wd3, bd3,
        W_emb3, b_emb3)
    U2, V2 = _node_precompute(node_type, emb_table, W_emb2, b_emb2)
    GhA, GhB = _sc_scatter(edge_index, WB, U2, V2)
    X = _node_finish(GhA, GhB, ln_g, ln_b, Ws0, bs0, Ws1, bs1, Wt0, Wt1, Wt2)
    return X, edge_feat
